# baseline (speedup 1.0000x reference)
import jax
import jax.numpy as jnp
from jax.experimental import pallas as pl
from jax.experimental.pallas import tpu as pltpu

N = 4096
D_IN = 512
D_OUT = 512

BI = 256
NI = N // BI


def _fused_kernel(x_ref, w_ref, adj_ref, o_ref, h_ref):
    s = pl.program_id(0)

    @pl.when(s == 0)
    def _build_h():
        h_ref[...] = jnp.dot(
            x_ref[...], w_ref[...], preferred_element_type=jnp.float32
        ).astype(jnp.bfloat16)

    @pl.when(s > 0)
    def _aggregate():
        a = adj_ref[...]
        deg = jnp.sum(a, axis=1, keepdims=True)
        acc = jnp.dot(
            a.astype(jnp.bfloat16), h_ref[...],
            preferred_element_type=jnp.float32,
        )
        o_ref[...] = acc / deg


@jax.jit
def kernel(input, adj, W):
    return pl.pallas_call(
        _fused_kernel,
        grid=(NI + 1,),
        in_specs=[
            pl.BlockSpec((N, D_IN), lambda s: (0, 0)),
            pl.BlockSpec((D_IN, D_OUT), lambda s: (0, 0)),
            pl.BlockSpec((BI, N), lambda s: (jnp.maximum(s - 1, 0), 0)),
        ],
        out_specs=pl.BlockSpec(
            (BI, D_OUT), lambda s: (jnp.maximum(s - 1, 0), 0)
        ),
        out_shape=jax.ShapeDtypeStruct((N, D_OUT), jnp.float32),
        scratch_shapes=[
            pltpu.VMEM((N, D_OUT), jnp.bfloat16),
        ],
        compiler_params=pltpu.CompilerParams(
            dimension_semantics=("arbitrary",),
        ),
    )(input, W, adj)


# speedup vs baseline: 1.1120x; 1.1120x over previous
import jax
import jax.numpy as jnp
from jax.experimental import pallas as pl
from jax.experimental.pallas import tpu as pltpu

N = 4096
D_IN = 512
D_OUT = 512

BI = 512
NI = N // BI


def _fused_kernel(x_ref, w_ref, adj_ref, o_ref, h_ref):
    s = pl.program_id(0)

    @pl.when(s == 0)
    def _build_h():
        h_ref[...] = jnp.dot(
            x_ref[...], w_ref[...], preferred_element_type=jnp.float32
        ).astype(jnp.bfloat16)

    @pl.when(s > 0)
    def _aggregate():
        a = adj_ref[...]
        deg = jnp.sum(a, axis=1, keepdims=True)
        acc = jnp.dot(
            a.astype(jnp.bfloat16), h_ref[...],
            preferred_element_type=jnp.float32,
        )
        o_ref[...] = acc / deg


@jax.jit
def kernel(input, adj, W):
    return pl.pallas_call(
        _fused_kernel,
        grid=(NI + 1,),
        in_specs=[
            pl.BlockSpec((N, D_IN), lambda s: (0, 0)),
            pl.BlockSpec((D_IN, D_OUT), lambda s: (0, 0)),
            pl.BlockSpec((BI, N), lambda s: (jnp.maximum(s - 1, 0), 0)),
        ],
        out_specs=pl.BlockSpec(
            (BI, D_OUT), lambda s: (jnp.maximum(s - 1, 0), 0)
        ),
        out_shape=jax.ShapeDtypeStruct((N, D_OUT), jnp.float32),
        scratch_shapes=[
            pltpu.VMEM((N, D_OUT), jnp.bfloat16),
        ],
        compiler_params=pltpu.CompilerParams(
            dimension_semantics=("arbitrary",),
        ),
    )(input, W, adj)
